# baseline (speedup 1.0000x reference)
import jax
import jax.numpy as jnp
from jax.experimental import pallas as pl
from jax.experimental.pallas import tpu as pltpu

_CHUNK_ROWS = (64, 128, 256, 512, 1152, 1024, 512, 256, 128, 64)
_OFFSETS = tuple(sum(_CHUNK_ROWS[:i]) for i in range(len(_CHUNK_ROWS)))
_N_CHUNKS = len(_CHUNK_ROWS)
_MAX_ROWS = max(_CHUNK_ROWS)


def _copy_kernel(x_ref, o_ref, scratch, in_sems, out_sems):
    for i in range(_N_CHUNKS):
        pltpu.make_async_copy(
            x_ref.at[pl.ds(_OFFSETS[i], _CHUNK_ROWS[i])],
            scratch.at[i, pl.ds(0, _CHUNK_ROWS[i])],
            in_sems.at[i],
        ).start()
    for i in range(_N_CHUNKS):
        pltpu.make_async_copy(
            x_ref.at[pl.ds(_OFFSETS[i], _CHUNK_ROWS[i])],
            scratch.at[i, pl.ds(0, _CHUNK_ROWS[i])],
            in_sems.at[i],
        ).wait()
        pltpu.make_async_copy(
            scratch.at[i, pl.ds(0, _CHUNK_ROWS[i])],
            o_ref.at[pl.ds(_OFFSETS[i], _CHUNK_ROWS[i])],
            out_sems.at[i],
        ).start()
    for i in range(_N_CHUNKS):
        pltpu.make_async_copy(
            scratch.at[i, pl.ds(0, _CHUNK_ROWS[i])],
            o_ref.at[pl.ds(_OFFSETS[i], _CHUNK_ROWS[i])],
            out_sems.at[i],
        ).wait()


def kernel(x, queue):
    del queue
    return pl.pallas_call(
        _copy_kernel,
        in_specs=[pl.BlockSpec(memory_space=pl.ANY)],
        out_specs=pl.BlockSpec(memory_space=pl.ANY),
        out_shape=jax.ShapeDtypeStruct(x.shape, x.dtype),
        scratch_shapes=[
            pltpu.VMEM((_N_CHUNKS, _MAX_ROWS, x.shape[1]), x.dtype),
            pltpu.SemaphoreType.DMA((_N_CHUNKS,)),
            pltpu.SemaphoreType.DMA((_N_CHUNKS,)),
        ],
    )(x)


# speedup vs baseline: 1.0233x; 1.0233x over previous
import jax
import jax.numpy as jnp
from jax.experimental import pallas as pl
from jax.experimental.pallas import tpu as pltpu

_N_CHUNKS = 8
_ROWS = 4096 // _N_CHUNKS


def _copy_kernel(x_ref, o_ref, scratch, in_sems, out_sems):
    for i in range(_N_CHUNKS):
        pltpu.make_async_copy(
            x_ref.at[pl.ds(i * _ROWS, _ROWS)], scratch.at[i], in_sems.at[i]
        ).start()
    for i in range(_N_CHUNKS):
        pltpu.make_async_copy(
            x_ref.at[pl.ds(i * _ROWS, _ROWS)], scratch.at[i], in_sems.at[i]
        ).wait()
        pltpu.make_async_copy(
            scratch.at[i], o_ref.at[pl.ds(i * _ROWS, _ROWS)], out_sems.at[i]
        ).start()
    for i in range(_N_CHUNKS):
        pltpu.make_async_copy(
            scratch.at[i], o_ref.at[pl.ds(i * _ROWS, _ROWS)], out_sems.at[i]
        ).wait()


def kernel(x, queue):
    del queue
    return pl.pallas_call(
        _copy_kernel,
        in_specs=[pl.BlockSpec(memory_space=pl.ANY)],
        out_specs=pl.BlockSpec(memory_space=pl.ANY),
        out_shape=jax.ShapeDtypeStruct(x.shape, x.dtype),
        scratch_shapes=[
            pltpu.VMEM((_N_CHUNKS, _ROWS, x.shape[1]), x.dtype),
            pltpu.SemaphoreType.DMA((_N_CHUNKS,)),
            pltpu.SemaphoreType.DMA((_N_CHUNKS,)),
        ],
    )(x)


# speedup vs baseline: 1.0406x; 1.0170x over previous
import jax
import jax.numpy as jnp
from jax.experimental import pallas as pl
from jax.experimental.pallas import tpu as pltpu

_N_CHUNKS = 4
_ROWS = 4096 // _N_CHUNKS


def _copy_kernel(x_ref, o_ref, scratch, in_sems, out_sems):
    for i in range(_N_CHUNKS):
        pltpu.make_async_copy(
            x_ref.at[pl.ds(i * _ROWS, _ROWS)], scratch.at[i], in_sems.at[i]
        ).start()
    for i in range(_N_CHUNKS):
        pltpu.make_async_copy(
            x_ref.at[pl.ds(i * _ROWS, _ROWS)], scratch.at[i], in_sems.at[i]
        ).wait()
        pltpu.make_async_copy(
            scratch.at[i], o_ref.at[pl.ds(i * _ROWS, _ROWS)], out_sems.at[i]
        ).start()
    for i in range(_N_CHUNKS):
        pltpu.make_async_copy(
            scratch.at[i], o_ref.at[pl.ds(i * _ROWS, _ROWS)], out_sems.at[i]
        ).wait()


def kernel(x, queue):
    del queue
    return pl.pallas_call(
        _copy_kernel,
        in_specs=[pl.BlockSpec(memory_space=pl.ANY)],
        out_specs=pl.BlockSpec(memory_space=pl.ANY),
        out_shape=jax.ShapeDtypeStruct(x.shape, x.dtype),
        scratch_shapes=[
            pltpu.VMEM((_N_CHUNKS, _ROWS, x.shape[1]), x.dtype),
            pltpu.SemaphoreType.DMA((_N_CHUNKS,)),
            pltpu.SemaphoreType.DMA((_N_CHUNKS,)),
        ],
    )(x)
